# baseline (speedup 1.0000x reference)
import jax
import jax.numpy as jnp
from jax import lax
from jax.experimental import pallas as pl
from jax.experimental.pallas import tpu as pltpu
from jax.experimental.pallas import tpu_sc as plsc

N = 10000
E = 160000
NT = 16
NC = 2
EPT = E // NT
B = 128
EH = 64
NB = 80
EPT_PAD = NB * B
E_PAD = NT * EPT_PAD
TRASH = N
ACC_STRIPE = 632
ACC_ROWS = NT * ACC_STRIPE
CW = 128
LPC = CW // 16
HCn = 1024 // CW
XCn = 256 // CW
K_TOT = 2 * XCn + 2 * HCn
D_GATE = 4096
NBLK = 1000


def _mm_t_body(a_ref, w_ref, b_ref, o_ref):
    o_ref[0] = (
        jnp.dot(a_ref[...], w_ref[0], preferred_element_type=jnp.float32)
        + b_ref[0, 0]
    )


def _mm_t(e_token, w_tok, b_link):
    return pl.pallas_call(
        _mm_t_body,
        grid=(N // NBLK, XCn),
        in_specs=[
            pl.BlockSpec((NBLK, 256), lambda i, c: (i, 0)),
            pl.BlockSpec((1, 256, CW), lambda i, c: (c, 0, 0)),
            pl.BlockSpec((1, 1, CW), lambda i, c: (c, 0, 0)),
        ],
        out_specs=pl.BlockSpec((1, NBLK, CW), lambda i, c: (c, i, 0)),
        out_shape=jax.ShapeDtypeStruct((XCn, N, CW), jnp.float32),
    )(e_token, w_tok.reshape(256, XCn, CW).transpose(1, 0, 2),
      b_link.reshape(XCn, 1, CW))


def _mm_el_body(a_ref, w_ref, o_ref):
    o_ref[0] = jnp.dot(a_ref[...], w_ref[0], preferred_element_type=jnp.float32)


def _mm_el(e_link_pad, w_el):
    eblk = 2048
    return pl.pallas_call(
        _mm_el_body,
        grid=(E_PAD // eblk, XCn),
        in_specs=[
            pl.BlockSpec((eblk, 128), lambda i, c: (i, 0)),
            pl.BlockSpec((1, 128, CW), lambda i, c: (c, 0, 0)),
        ],
        out_specs=pl.BlockSpec((1, eblk, CW), lambda i, c: (c, i, 0)),
        out_shape=jax.ShapeDtypeStruct((XCn, E_PAD, CW), jnp.float32),
    )(e_link_pad, w_el.reshape(128, XCn, CW).transpose(1, 0, 2))


def _sc_body(h2, t2, el_t, fg, fs, tg, ts, out,
             idxb0, idxb1, igrow, iscrow0, iscrow1, bufA0, bufA1, bufEl, acc,
             semA, semB, semS0, semS1):
    cid = lax.axis_index("c")
    sid = lax.axis_index("s")
    idxbs = (idxb0, idxb1)
    iscs = (iscrow0, iscrow1)
    bufAs = (bufA0, bufA1)
    sems = (semA, semB)
    ssems = (semS0, semS1)

    def zero_bufEl():
        def _zrow(r, carry):
            for cc in range(LPC):
                bufEl[r, pl.ds(cc * 16, 16)] = jnp.zeros((16,), jnp.float32)
            return carry
        lax.fori_loop(0, EH, _zrow, 0)

    def zero_acc():
        base = sid * ACC_STRIPE
        nfull = ACC_STRIPE // EH
        for q in range(nfull):
            pltpu.sync_copy(bufEl, acc.at[pl.ds(base + q * EH, EH)])
        rem = ACC_STRIPE % EH
        if rem:
            pltpu.sync_copy(bufEl.at[pl.ds(0, rem)],
                            acc.at[pl.ds(base + nfull * EH, rem)])

    def flush(kk):
        lo = sid * ACC_STRIPE

        @pl.when(sid != NT - 1)
        def _():
            pltpu.sync_copy(acc.at[pl.ds(lo, ACC_STRIPE)],
                            out.at[kk, pl.ds(lo, ACC_STRIPE)])

        @pl.when(sid == NT - 1)
        def _():
            last = N - (NT - 1) * ACC_STRIPE
            pltpu.sync_copy(acc.at[pl.ds((NT - 1) * ACC_STRIPE, last)],
                            out.at[kk, pl.ds((NT - 1) * ACC_STRIPE, last)])

    def fetch_and_start(tab, ig, bb, q, mult, off):
        pltpu.sync_copy(ig.at[sid, bb], igrow.at[0])
        for cc in range(B // 16):
            v = igrow[0, pl.ds(cc * 16, 16)]
            idxbs[q][0, pl.ds(cc * 16, 16)] = v * mult + off
        pltpu.make_async_copy(tab.at[idxbs[q].at[0]], bufAs[q], sems[q]).start()

    def gather_wait(tab, q):
        pltpu.make_async_copy(tab.at[idxbs[q].at[0]], bufAs[q], sems[q]).wait()

    def scatter_start(q):
        pltpu.make_async_copy(
            bufAs[q], acc.at[iscs[q].at[0]], ssems[q]).start(add=True)

    def scatter_wait(p):
        pltpu.make_async_copy(
            bufAs[p], acc.at[iscs[p].at[0]], ssems[p]).wait()

    def do_pass(kk, tab, ig, isc, mult, off, xc):
        zero_acc()
        plsc.subcore_barrier()
        fetch_and_start(tab, ig, 0, 0, mult, off)

        def step(i, carry):
            for q in range(2):
                bb = 2 * i + q
                gather_wait(tab, q)
                if xc is None:
                    if q == 1:
                        scatter_wait(0)
                    else:
                        @pl.when(i > 0)
                        def _():
                            scatter_wait(1)

                @pl.when(bb + 1 < NB)
                def _():
                    fetch_and_start(tab, ig, bb + 1, 1 - q, mult, off)

                pltpu.sync_copy(isc.at[sid, bb], iscs[q].at[0])
                if xc is None:
                    scatter_start(q)
                else:
                    for hh in range(B // EH):
                        pltpu.sync_copy(
                            el_t.at[xc, pl.ds(sid * EPT_PAD + bb * B + hh * EH,
                                              EH)], bufEl)

                        def trow(r, c2):
                            for cc in range(LPC):
                                sl = pl.ds(cc * 16, 16)
                                v = bufEl[r, sl] + bufAs[q][hh * EH + r, sl]
                                ex = jnp.exp(v * 2.0)
                                bufEl[r, sl] = 1.0 - 2.0 / (ex + 1.0)
                            return c2
                        lax.fori_loop(0, EH, trow, 0)
                        pltpu.sync_copy(
                            bufEl,
                            acc.at[iscs[q].at[0, pl.ds(hh * EH, EH)]], add=True)
            return carry
        lax.fori_loop(0, NB // 2, step, 0)
        if xc is None:
            scatter_wait(1)
        plsc.subcore_barrier()
        flush(kk)
        return 0

    zero_bufEl()

    def h_in(p, c):
        chunk = cid * (HCn // NC) + p
        return do_pass(2 * XCn + chunk, h2, fg, ts, HCn, chunk, None)

    def h_out(p, c):
        chunk = cid * (HCn // NC) + p
        return do_pass(2 * XCn + HCn + chunk, h2, tg, fs, HCn, chunk, None)

    lax.fori_loop(0, HCn // NC, h_in, 0)
    lax.fori_loop(0, HCn // NC, h_out, 0)
    for p in range(XCn // NC):
        xc = cid * (XCn // NC) + p
        do_pass(xc, t2, fg, ts, 1, xc * N, xc)
        zero_bufEl()
        do_pass(XCn + xc, t2, fg, fs, 1, xc * N, xc)
        zero_bufEl()


def _seg_sums_sc(h2, t2, el_t, fg, fs, tg, ts):
    mesh = plsc.VectorSubcoreMesh(core_axis_name="c", subcore_axis_name="s")
    return pl.kernel(
        _sc_body,
        out_type=jax.ShapeDtypeStruct((K_TOT, N, CW), jnp.float32),
        mesh=mesh,
        scratch_types=[
            pltpu.VMEM((1, B), jnp.int32),
            pltpu.VMEM((1, B), jnp.int32),
            pltpu.VMEM((1, B), jnp.int32),
            pltpu.VMEM((1, B), jnp.int32),
            pltpu.VMEM((1, B), jnp.int32),
            pltpu.VMEM((B, CW), jnp.float32),
            pltpu.VMEM((B, CW), jnp.float32),
            pltpu.VMEM((EH, CW), jnp.float32),
            pltpu.VMEM_SHARED((ACC_ROWS, CW), jnp.float32),
            pltpu.SemaphoreType.DMA,
            pltpu.SemaphoreType.DMA,
            pltpu.SemaphoreType.DMA,
            pltpu.SemaphoreType.DMA,
        ],
    )(h2, t2, el_t, fg, fs, tg, ts)


def _gate_body(inp_ref, w_ref, b_ref, c_ref, h_out, c_out, acc):
    k = pl.program_id(1)

    @pl.when(k == 0)
    def _():
        acc[...] = jnp.zeros_like(acc)

    acc[...] += jnp.dot(inp_ref[0], w_ref[0], preferred_element_type=jnp.float32)

    @pl.when(k == K_TOT - 1)
    def _():
        g = acc[...] + b_ref[...]
        gi = g[:, 0:1024]
        go = g[:, 1024:2048]
        gf = g[:, 2048:3072]
        gu = g[:, 3072:4096]
        si = 1.0 / (1.0 + jnp.exp(-gi))
        so = 1.0 / (1.0 + jnp.exp(-go))
        sf = 1.0 / (1.0 + jnp.exp(-gf))
        u = jnp.tanh(gu)
        c2 = sf * c_ref[...] + si * u
        c_out[...] = c2
        h_out[...] = so * jnp.tanh(c2)


def _gates(inp_t, w_all, b_all, c_node):
    return pl.pallas_call(
        _gate_body,
        grid=(N // NBLK, K_TOT),
        in_specs=[
            pl.BlockSpec((1, NBLK, CW), lambda i, k: (k, i, 0)),
            pl.BlockSpec((1, CW, D_GATE), lambda i, k: (k, 0, 0)),
            pl.BlockSpec((1, D_GATE), lambda i, k: (0, 0)),
            pl.BlockSpec((NBLK, 1024), lambda i, k: (i, 0)),
        ],
        out_specs=[
            pl.BlockSpec((NBLK, 1024), lambda i, k: (i, 0)),
            pl.BlockSpec((NBLK, 1024), lambda i, k: (i, 0)),
        ],
        out_shape=[
            jax.ShapeDtypeStruct((N, 1024), jnp.float32),
            jax.ShapeDtypeStruct((N, 1024), jnp.float32),
        ],
        scratch_shapes=[pltpu.VMEM((NBLK, D_GATE), jnp.float32)],
    )(inp_t, w_all, b_all, c_node)


def kernel(h_node, c_node, e_link, e_token, i_from, i_to,
           W_link, b_link, W_i, b_i, W_o, b_o, W_f, b_f, W_u, b_u):
    w_el = W_link[:128]
    w_tok = W_link[128:]
    w_all = jnp.concatenate([W_i, W_o, W_f, W_u], axis=1).reshape(K_TOT, CW, D_GATE)
    b_all = jnp.concatenate([b_i, b_o, b_f, b_u]).reshape(1, D_GATE)

    def pad_to(ix, dummy):
        a2 = ix.reshape(NT, EPT)
        pad = jnp.full((NT, EPT_PAD - EPT), dummy, jnp.int32)
        return jnp.concatenate([a2, pad], axis=1).reshape(NT, NB, B)

    fg = pad_to(i_from, 0)
    tg = pad_to(i_to, 0)
    fs = pad_to(i_from, TRASH)
    ts = pad_to(i_to, TRASH)

    e_link_pad = jnp.pad(
        e_link.reshape(NT, EPT, 128), ((0, 0), (0, EPT_PAD - EPT), (0, 0))
    ).reshape(E_PAD, 128)

    t2 = _mm_t(e_token, w_tok, b_link).reshape(N * XCn, CW)
    el_t = _mm_el(e_link_pad, w_el)
    h2 = h_node.reshape(N * HCn, CW)

    inp_t = _seg_sums_sc(h2, t2, el_t, fg, fs, tg, ts)
    h_new, c_new = _gates(inp_t, w_all, b_all, c_node)
    return h_new, c_new


# speedup vs baseline: 1.0294x; 1.0294x over previous
import jax
import jax.numpy as jnp
from jax import lax
from jax.experimental import pallas as pl
from jax.experimental.pallas import tpu as pltpu
from jax.experimental.pallas import tpu_sc as plsc

N = 10000
E = 160000
NT = 16
NC = 2
EPT = E // NT
B = 128
EH = 64
NB = 80
G = 8
EPT_PAD = NB * B
E_PAD = NT * EPT_PAD
TRASH = N
ACC_STRIPE = 632
ACC_ROWS = NT * ACC_STRIPE
CW = 128
LPC = CW // 16
HCn = 1024 // CW
XCn = 256 // CW
K_TOT = 2 * XCn + 2 * HCn
D_GATE = 4096
NBLK = 1000


def _mm_t_body(a_ref, w_ref, b_ref, o_ref):
    o_ref[0] = (
        jnp.dot(a_ref[...], w_ref[0], preferred_element_type=jnp.float32)
        + b_ref[0, 0]
    )


def _mm_t(e_token, w_tok, b_link):
    return pl.pallas_call(
        _mm_t_body,
        grid=(N // NBLK, XCn),
        in_specs=[
            pl.BlockSpec((NBLK, 256), lambda i, c: (i, 0)),
            pl.BlockSpec((1, 256, CW), lambda i, c: (c, 0, 0)),
            pl.BlockSpec((1, 1, CW), lambda i, c: (c, 0, 0)),
        ],
        out_specs=pl.BlockSpec((1, NBLK, CW), lambda i, c: (c, i, 0)),
        out_shape=jax.ShapeDtypeStruct((XCn, N, CW), jnp.float32),
    )(e_token, w_tok.reshape(256, XCn, CW).transpose(1, 0, 2),
      b_link.reshape(XCn, 1, CW))


def _mm_el_body(a_ref, w_ref, o_ref):
    o_ref[0] = jnp.dot(a_ref[...], w_ref[0], preferred_element_type=jnp.float32)


def _mm_el(e_link_pad, w_el):
    eblk = 2048
    return pl.pallas_call(
        _mm_el_body,
        grid=(E_PAD // eblk, XCn),
        in_specs=[
            pl.BlockSpec((eblk, 128), lambda i, c: (i, 0)),
            pl.BlockSpec((1, 128, CW), lambda i, c: (c, 0, 0)),
        ],
        out_specs=pl.BlockSpec((1, eblk, CW), lambda i, c: (c, i, 0)),
        out_shape=jax.ShapeDtypeStruct((XCn, E_PAD, CW), jnp.float32),
    )(e_link_pad, w_el.reshape(128, XCn, CW).transpose(1, 0, 2))


def _sc_body(h2, t2, el_t, fg, fs, tg, ts, out,
             igblk, iscblk, bufA0, bufA1, bufEl, acc,
             semA, semB, semS0, semS1):
    cid = lax.axis_index("c")
    sid = lax.axis_index("s")
    bufAs = (bufA0, bufA1)
    sems = (semA, semB)
    ssems = (semS0, semS1)

    def zero_bufEl():
        def _zrow(r, carry):
            for cc in range(LPC):
                bufEl[r, pl.ds(cc * 16, 16)] = jnp.zeros((16,), jnp.float32)
            return carry
        lax.fori_loop(0, EH, _zrow, 0)

    def zero_acc():
        base = sid * ACC_STRIPE
        nfull = ACC_STRIPE // EH
        for q in range(nfull):
            pltpu.sync_copy(bufEl, acc.at[pl.ds(base + q * EH, EH)])
        rem = ACC_STRIPE % EH
        if rem:
            pltpu.sync_copy(bufEl.at[pl.ds(0, rem)],
                            acc.at[pl.ds(base + nfull * EH, rem)])

    def flush(kk):
        lo = sid * ACC_STRIPE

        @pl.when(sid != NT - 1)
        def _():
            pltpu.sync_copy(acc.at[pl.ds(lo, ACC_STRIPE)],
                            out.at[kk, pl.ds(lo, ACC_STRIPE)])

        @pl.when(sid == NT - 1)
        def _():
            last = N - (NT - 1) * ACC_STRIPE
            pltpu.sync_copy(acc.at[pl.ds((NT - 1) * ACC_STRIPE, last)],
                            out.at[kk, pl.ds((NT - 1) * ACC_STRIPE, last)])

    def fetch_and_start(tab, r, q):
        pltpu.make_async_copy(tab.at[igblk.at[r]], bufAs[q], sems[q]).start()

    def gather_wait(tab, r, q):
        pltpu.make_async_copy(tab.at[igblk.at[r]], bufAs[q], sems[q]).wait()

    def scatter_start(r, q):
        pltpu.make_async_copy(
            bufAs[q], acc.at[iscblk.at[r]], ssems[q]).start(add=True)

    def scatter_wait(r, p):
        pltpu.make_async_copy(
            bufAs[p], acc.at[iscblk.at[r]], ssems[p]).wait()

    def do_pass(kk, tab, ig, isc, mult, off, xc):
        zero_acc()
        plsc.subcore_barrier()

        def blkbody(blk, carry):
            pltpu.sync_copy(ig.at[sid, pl.ds(blk * G, G)], igblk)
            pltpu.sync_copy(isc.at[sid, pl.ds(blk * G, G)], iscblk)

            def _scale(r, c):
                for cc in range(B // 16):
                    sl = pl.ds(cc * 16, 16)
                    igblk[r, sl] = igblk[r, sl] * mult + off
                return c
            lax.fori_loop(0, G, _scale, 0)
            fetch_and_start(tab, 0, 0)

            def step(i, carry2):
                for q in range(2):
                    r = 2 * i + q
                    gather_wait(tab, r, q)
                    if xc is None:
                        if q == 1:
                            scatter_wait(2 * i, 0)
                        else:
                            @pl.when(i > 0)
                            def _():
                                scatter_wait(2 * i - 1, 1)

                    @pl.when(r + 1 < G)
                    def _():
                        fetch_and_start(tab, r + 1, 1 - q)

                    if xc is None:
                        scatter_start(r, q)
                    else:
                        bb = blk * G + r
                        for hh in range(B // EH):
                            pltpu.sync_copy(
                                el_t.at[xc, pl.ds(
                                    sid * EPT_PAD + bb * B + hh * EH, EH)],
                                bufEl)

                            def trow(rr, c2):
                                for cc in range(LPC):
                                    sl = pl.ds(cc * 16, 16)
                                    v = bufEl[rr, sl] + bufAs[q][hh * EH + rr, sl]
                                    ex = jnp.exp(v * 2.0)
                                    bufEl[rr, sl] = 1.0 - 2.0 / (ex + 1.0)
                                return c2
                            lax.fori_loop(0, EH, trow, 0)
                            pltpu.sync_copy(
                                bufEl,
                                acc.at[iscblk.at[r, pl.ds(hh * EH, EH)]],
                                add=True)
                return carry2
            lax.fori_loop(0, G // 2, step, 0)
            if xc is None:
                scatter_wait(G - 1, 1)
            return carry
        lax.fori_loop(0, NB // G, blkbody, 0)
        plsc.subcore_barrier()
        flush(kk)
        return 0

    zero_bufEl()

    def h_in(p, c):
        chunk = cid * (HCn // NC) + p
        return do_pass(2 * XCn + chunk, h2, fg, ts, HCn, chunk, None)

    def h_out(p, c):
        chunk = cid * (HCn // NC) + p
        return do_pass(2 * XCn + HCn + chunk, h2, tg, fs, HCn, chunk, None)

    lax.fori_loop(0, HCn // NC, h_in, 0)
    lax.fori_loop(0, HCn // NC, h_out, 0)
    for p in range(XCn // NC):
        xc = cid * (XCn // NC) + p
        do_pass(xc, t2, fg, ts, 1, xc * N, xc)
        zero_bufEl()
        do_pass(XCn + xc, t2, fg, fs, 1, xc * N, xc)
        zero_bufEl()


def _seg_sums_sc(h2, t2, el_t, fg, fs, tg, ts):
    mesh = plsc.VectorSubcoreMesh(core_axis_name="c", subcore_axis_name="s")
    return pl.kernel(
        _sc_body,
        out_type=jax.ShapeDtypeStruct((K_TOT, N, CW), jnp.float32),
        mesh=mesh,
        scratch_types=[
            pltpu.VMEM((G, B), jnp.int32),
            pltpu.VMEM((G, B), jnp.int32),
            pltpu.VMEM((B, CW), jnp.float32),
            pltpu.VMEM((B, CW), jnp.float32),
            pltpu.VMEM((EH, CW), jnp.float32),
            pltpu.VMEM_SHARED((ACC_ROWS, CW), jnp.float32),
            pltpu.SemaphoreType.DMA,
            pltpu.SemaphoreType.DMA,
            pltpu.SemaphoreType.DMA,
            pltpu.SemaphoreType.DMA,
        ],
    )(h2, t2, el_t, fg, fs, tg, ts)


def _gate_body(inp_ref, w_ref, b_ref, c_ref, h_out, c_out, acc):
    k = pl.program_id(1)

    @pl.when(k == 0)
    def _():
        acc[...] = jnp.zeros_like(acc)

    acc[...] += jnp.dot(inp_ref[0], w_ref[0], preferred_element_type=jnp.float32)

    @pl.when(k == K_TOT - 1)
    def _():
        g = acc[...] + b_ref[...]
        gi = g[:, 0:1024]
        go = g[:, 1024:2048]
        gf = g[:, 2048:3072]
        gu = g[:, 3072:4096]
        si = 1.0 / (1.0 + jnp.exp(-gi))
        so = 1.0 / (1.0 + jnp.exp(-go))
        sf = 1.0 / (1.0 + jnp.exp(-gf))
        u = jnp.tanh(gu)
        c2 = sf * c_ref[...] + si * u
        c_out[...] = c2
        h_out[...] = so * jnp.tanh(c2)


def _gates(inp_t, w_all, b_all, c_node):
    return pl.pallas_call(
        _gate_body,
        grid=(N // NBLK, K_TOT),
        in_specs=[
            pl.BlockSpec((1, NBLK, CW), lambda i, k: (k, i, 0)),
            pl.BlockSpec((1, CW, D_GATE), lambda i, k: (k, 0, 0)),
            pl.BlockSpec((1, D_GATE), lambda i, k: (0, 0)),
            pl.BlockSpec((NBLK, 1024), lambda i, k: (i, 0)),
        ],
        out_specs=[
            pl.BlockSpec((NBLK, 1024), lambda i, k: (i, 0)),
            pl.BlockSpec((NBLK, 1024), lambda i, k: (i, 0)),
        ],
        out_shape=[
            jax.ShapeDtypeStruct((N, 1024), jnp.float32),
            jax.ShapeDtypeStruct((N, 1024), jnp.float32),
        ],
        scratch_shapes=[pltpu.VMEM((NBLK, D_GATE), jnp.float32)],
    )(inp_t, w_all, b_all, c_node)


def kernel(h_node, c_node, e_link, e_token, i_from, i_to,
           W_link, b_link, W_i, b_i, W_o, b_o, W_f, b_f, W_u, b_u):
    w_el = W_link[:128]
    w_tok = W_link[128:]
    w_all = jnp.concatenate([W_i, W_o, W_f, W_u], axis=1).reshape(K_TOT, CW, D_GATE)
    b_all = jnp.concatenate([b_i, b_o, b_f, b_u]).reshape(1, D_GATE)

    def pad_to(ix, dummy):
        a2 = ix.reshape(NT, EPT)
        pad = jnp.full((NT, EPT_PAD - EPT), dummy, jnp.int32)
        return jnp.concatenate([a2, pad], axis=1).reshape(NT, NB, B)

    fg = pad_to(i_from, 0)
    tg = pad_to(i_to, 0)
    fs = pad_to(i_from, TRASH)
    ts = pad_to(i_to, TRASH)

    e_link_pad = jnp.pad(
        e_link.reshape(NT, EPT, 128), ((0, 0), (0, EPT_PAD - EPT), (0, 0))
    ).reshape(E_PAD, 128)

    t2 = _mm_t(e_token, w_tok, b_link).reshape(N * XCn, CW)
    el_t = _mm_el(e_link_pad, w_el)
    h2 = h_node.reshape(N * HCn, CW)

    inp_t = _seg_sums_sc(h2, t2, el_t, fg, fs, tg, ts)
    h_new, c_new = _gates(inp_t, w_all, b_all, c_node)
    return h_new, c_new


# speedup vs baseline: 1.1523x; 1.1195x over previous
import jax
import jax.numpy as jnp
from jax import lax
from jax.experimental import pallas as pl
from jax.experimental.pallas import tpu as pltpu
from jax.experimental.pallas import tpu_sc as plsc

N = 10000
E = 160000
NT = 16
NC = 2
EPT = E // NT
B = 128
EH = 64
NB = 80
G = 8
EPT_PAD = NB * B
E_PAD = NT * EPT_PAD
TRASH = N
ACC_STRIPE = 632
ACC_ROWS = NT * ACC_STRIPE
CW = 128
LPC = CW // 16
HCn = 1024 // CW
XCn = 256 // CW
K_TOT = 2 * XCn + 2 * HCn
D_GATE = 4096
NBLK = 1000


def _mm_t_body(a_ref, w_ref, b_ref, o_ref):
    o_ref[0] = (
        jnp.dot(a_ref[...], w_ref[0], preferred_element_type=jnp.float32)
        + b_ref[0, 0]
    )


def _mm_t(e_token, w_tok, b_link):
    return pl.pallas_call(
        _mm_t_body,
        grid=(N // NBLK, XCn),
        in_specs=[
            pl.BlockSpec((NBLK, 256), lambda i, c: (i, 0)),
            pl.BlockSpec((1, 256, CW), lambda i, c: (c, 0, 0)),
            pl.BlockSpec((1, 1, CW), lambda i, c: (c, 0, 0)),
        ],
        out_specs=pl.BlockSpec((1, NBLK, CW), lambda i, c: (c, i, 0)),
        out_shape=jax.ShapeDtypeStruct((XCn, N, CW), jnp.float32),
    )(e_token, w_tok.reshape(256, XCn, CW).transpose(1, 0, 2),
      b_link.reshape(XCn, 1, CW))


def _mm_el_body(a_ref, w_ref, o_ref):
    o_ref[0] = jnp.dot(a_ref[...], w_ref[0], preferred_element_type=jnp.float32)


def _mm_el(e_link_pad, w_el):
    eblk = 2048
    return pl.pallas_call(
        _mm_el_body,
        grid=(E_PAD // eblk, XCn),
        in_specs=[
            pl.BlockSpec((eblk, 128), lambda i, c: (i, 0)),
            pl.BlockSpec((1, 128, CW), lambda i, c: (c, 0, 0)),
        ],
        out_specs=pl.BlockSpec((1, eblk, CW), lambda i, c: (c, i, 0)),
        out_shape=jax.ShapeDtypeStruct((XCn, E_PAD, CW), jnp.float32),
    )(e_link_pad, w_el.reshape(128, XCn, CW).transpose(1, 0, 2))


def _sc_core(mode, h2, t2, el_t, fg, fs, tg, ts, out,
             igblk, iscblk, bufA0, bufA1, bufEl, acc,
             semA, semB, semS0, semS1):
    cid = lax.axis_index("c")
    sid = lax.axis_index("s")
    bufAs = (bufA0, bufA1)
    sems = (semA, semB)
    ssems = (semS0, semS1)

    def zero_bufEl():
        def _zrow(r, carry):
            for cc in range(LPC):
                bufEl[r, pl.ds(cc * 16, 16)] = jnp.zeros((16,), jnp.float32)
            return carry
        lax.fori_loop(0, EH, _zrow, 0)

    def zero_acc():
        base = sid * ACC_STRIPE
        nfull = ACC_STRIPE // EH
        for q in range(nfull):
            pltpu.sync_copy(bufEl, acc.at[pl.ds(base + q * EH, EH)])
        rem = ACC_STRIPE % EH
        if rem:
            pltpu.sync_copy(bufEl.at[pl.ds(0, rem)],
                            acc.at[pl.ds(base + nfull * EH, rem)])

    def flush(kk):
        lo = sid * ACC_STRIPE

        @pl.when(sid != NT - 1)
        def _():
            pltpu.sync_copy(acc.at[pl.ds(lo, ACC_STRIPE)],
                            out.at[kk, pl.ds(lo, ACC_STRIPE)])

        @pl.when(sid == NT - 1)
        def _():
            last = N - (NT - 1) * ACC_STRIPE
            pltpu.sync_copy(acc.at[pl.ds((NT - 1) * ACC_STRIPE, last)],
                            out.at[kk, pl.ds((NT - 1) * ACC_STRIPE, last)])

    def fetch_and_start(tab, r, q):
        pltpu.make_async_copy(tab.at[igblk.at[r]], bufAs[q], sems[q]).start()

    def gather_wait(tab, r, q):
        pltpu.make_async_copy(tab.at[igblk.at[r]], bufAs[q], sems[q]).wait()

    def scatter_start(r, q):
        pltpu.make_async_copy(
            bufAs[q], acc.at[iscblk.at[r]], ssems[q]).start(add=True)

    def scatter_wait(r, p):
        pltpu.make_async_copy(
            bufAs[p], acc.at[iscblk.at[r]], ssems[p]).wait()

    def do_pass(kk, tab, ig, isc, mult, off, xc):
        zero_acc()
        plsc.subcore_barrier()

        def blkbody(blk, carry):
            pltpu.sync_copy(ig.at[sid, pl.ds(blk * G, G)], igblk)
            pltpu.sync_copy(isc.at[sid, pl.ds(blk * G, G)], iscblk)

            def _scale(r, c):
                for cc in range(B // 16):
                    sl = pl.ds(cc * 16, 16)
                    igblk[r, sl] = igblk[r, sl] * mult + off
                return c
            lax.fori_loop(0, G, _scale, 0)
            fetch_and_start(tab, 0, 0)

            def step(i, carry2):
                for q in range(2):
                    r = 2 * i + q
                    gather_wait(tab, r, q)
                    if xc is None:
                        if q == 1:
                            scatter_wait(2 * i, 0)
                        else:
                            @pl.when(i > 0)
                            def _():
                                scatter_wait(2 * i - 1, 1)

                    @pl.when(r + 1 < G)
                    def _():
                        fetch_and_start(tab, r + 1, 1 - q)

                    if xc is None:
                        scatter_start(r, q)
                    else:
                        bb = blk * G + r
                        for hh in range(B // EH):
                            pltpu.sync_copy(
                                el_t.at[xc, pl.ds(
                                    sid * EPT_PAD + bb * B + hh * EH, EH)],
                                bufEl)

                            def trow(rr, c2):
                                for cc in range(LPC):
                                    sl = pl.ds(cc * 16, 16)
                                    v = bufEl[rr, sl] + bufAs[q][hh * EH + rr, sl]
                                    ex = jnp.exp(v * 2.0)
                                    bufEl[rr, sl] = 1.0 - 2.0 / (ex + 1.0)
                                return c2
                            lax.fori_loop(0, EH, trow, 0)
                            pltpu.sync_copy(
                                bufEl,
                                acc.at[iscblk.at[r, pl.ds(hh * EH, EH)]],
                                add=True)
                return carry2
            lax.fori_loop(0, G // 2, step, 0)
            if xc is None:
                scatter_wait(G - 1, 1)
            return carry
        lax.fori_loop(0, NB // G, blkbody, 0)
        plsc.subcore_barrier()
        flush(kk)
        return 0

    zero_bufEl()
    if mode == "h":
        def h_in(p, c):
            chunk = cid * (HCn // NC) + p
            return do_pass(chunk, h2, fg, ts, HCn, chunk, None)

        def h_out(p, c):
            chunk = cid * (HCn // NC) + p
            return do_pass(HCn + chunk, h2, tg, fs, HCn, chunk, None)

        lax.fori_loop(0, HCn // NC, h_in, 0)
        lax.fori_loop(0, HCn // NC, h_out, 0)
    else:
        for p in range(XCn // NC):
            xc = cid * (XCn // NC) + p
            do_pass(xc, t2, fg, ts, 1, xc * N, xc)
            zero_bufEl()
            do_pass(XCn + xc, t2, fg, fs, 1, xc * N, xc)
            zero_bufEl()


def _sc_body_h(h2, fg, fs, tg, ts, out, *rest):
    _sc_core("h", h2, None, None, fg, fs, tg, ts, out, *rest)


def _sc_body_x(t2, el_t, fg, fs, ts, out, *rest):
    _sc_core("x", None, t2, el_t, fg, fs, None, ts, out, *rest)


def _sc_scratch():
    return [
        pltpu.VMEM((G, B), jnp.int32),
        pltpu.VMEM((G, B), jnp.int32),
        pltpu.VMEM((B, CW), jnp.float32),
        pltpu.VMEM((B, CW), jnp.float32),
        pltpu.VMEM((EH, CW), jnp.float32),
        pltpu.VMEM_SHARED((ACC_ROWS, CW), jnp.float32),
        pltpu.SemaphoreType.DMA,
        pltpu.SemaphoreType.DMA,
        pltpu.SemaphoreType.DMA,
        pltpu.SemaphoreType.DMA,
    ]


def _seg_h(h2, fg, fs, tg, ts):
    mesh = plsc.VectorSubcoreMesh(core_axis_name="c", subcore_axis_name="s")
    return pl.kernel(
        _sc_body_h,
        out_type=jax.ShapeDtypeStruct((2 * HCn, N, CW), jnp.float32),
        mesh=mesh,
        scratch_types=_sc_scratch(),
    )(h2, fg, fs, tg, ts)


def _seg_x(t2, el_t, fg, fs, ts):
    mesh = plsc.VectorSubcoreMesh(core_axis_name="c", subcore_axis_name="s")
    return pl.kernel(
        _sc_body_x,
        out_type=jax.ShapeDtypeStruct((2 * XCn, N, CW), jnp.float32),
        mesh=mesh,
        scratch_types=_sc_scratch(),
    )(t2, el_t, fg, fs, ts)


def _gate_h_body(inp_ref, w_ref, part_out, acc):
    k = pl.program_id(1)

    @pl.when(k == 0)
    def _():
        acc[...] = jnp.zeros_like(acc)

    acc[...] += jnp.dot(inp_ref[0], w_ref[0], preferred_element_type=jnp.float32)

    @pl.when(k == 2 * HCn - 1)
    def _():
        part_out[...] = acc[...]


def _gates_partial_h(inp_h, w_h):
    return pl.pallas_call(
        _gate_h_body,
        grid=(N // NBLK, 2 * HCn),
        in_specs=[
            pl.BlockSpec((1, NBLK, CW), lambda i, k: (k, i, 0)),
            pl.BlockSpec((1, CW, D_GATE), lambda i, k: (k, 0, 0)),
        ],
        out_specs=pl.BlockSpec((NBLK, D_GATE), lambda i, k: (i, 0)),
        out_shape=jax.ShapeDtypeStruct((N, D_GATE), jnp.float32),
        scratch_shapes=[pltpu.VMEM((NBLK, D_GATE), jnp.float32)],
    )(inp_h, w_h)


def _gate_x_body(inp_ref, w_ref, part_ref, b_ref, c_ref, h_out, c_out, acc):
    k = pl.program_id(1)

    @pl.when(k == 0)
    def _():
        acc[...] = part_ref[...]

    acc[...] += jnp.dot(inp_ref[0], w_ref[0], preferred_element_type=jnp.float32)

    @pl.when(k == 2 * XCn - 1)
    def _():
        g = acc[...] + b_ref[...]
        gi = g[:, 0:1024]
        go = g[:, 1024:2048]
        gf = g[:, 2048:3072]
        gu = g[:, 3072:4096]
        si = 1.0 / (1.0 + jnp.exp(-gi))
        so = 1.0 / (1.0 + jnp.exp(-go))
        sf = 1.0 / (1.0 + jnp.exp(-gf))
        u = jnp.tanh(gu)
        c2 = sf * c_ref[...] + si * u
        c_out[...] = c2
        h_out[...] = so * jnp.tanh(c2)


def _gates_final_x(inp_x, w_x, part, b_all, c_node):
    nb = 400
    return pl.pallas_call(
        _gate_x_body,
        grid=(N // nb, 2 * XCn),
        in_specs=[
            pl.BlockSpec((1, nb, CW), lambda i, k: (k, i, 0)),
            pl.BlockSpec((1, CW, D_GATE), lambda i, k: (k, 0, 0)),
            pl.BlockSpec((nb, D_GATE), lambda i, k: (i, 0)),
            pl.BlockSpec((1, D_GATE), lambda i, k: (0, 0)),
            pl.BlockSpec((nb, 1024), lambda i, k: (i, 0)),
        ],
        out_specs=[
            pl.BlockSpec((nb, 1024), lambda i, k: (i, 0)),
            pl.BlockSpec((nb, 1024), lambda i, k: (i, 0)),
        ],
        out_shape=[
            jax.ShapeDtypeStruct((N, 1024), jnp.float32),
            jax.ShapeDtypeStruct((N, 1024), jnp.float32),
        ],
        scratch_shapes=[pltpu.VMEM((nb, D_GATE), jnp.float32)],
    )(inp_x, w_x, part, b_all, c_node)


def kernel(h_node, c_node, e_link, e_token, i_from, i_to,
           W_link, b_link, W_i, b_i, W_o, b_o, W_f, b_f, W_u, b_u):
    w_el = W_link[:128]
    w_tok = W_link[128:]
    w_all = jnp.concatenate([W_i, W_o, W_f, W_u], axis=1).reshape(K_TOT, CW, D_GATE)
    w_x = w_all[:2 * XCn]
    w_h = w_all[2 * XCn:]
    b_all = jnp.concatenate([b_i, b_o, b_f, b_u]).reshape(1, D_GATE)

    def pad_to(ix, dummy):
        a2 = ix.reshape(NT, EPT)
        pad = jnp.full((NT, EPT_PAD - EPT), dummy, jnp.int32)
        return jnp.concatenate([a2, pad], axis=1).reshape(NT, NB, B)

    fg = pad_to(i_from, 0)
    tg = pad_to(i_to, 0)
    fs = pad_to(i_from, TRASH)
    ts = pad_to(i_to, TRASH)

    e_link_pad = jnp.pad(
        e_link.reshape(NT, EPT, 128), ((0, 0), (0, EPT_PAD - EPT), (0, 0))
    ).reshape(E_PAD, 128)

    h2 = h_node.reshape(N * HCn, CW)
    inp_h = _seg_h(h2, fg, fs, tg, ts)
    t2 = _mm_t(e_token, w_tok, b_link).reshape(N * XCn, CW)
    el_t = _mm_el(e_link_pad, w_el)
    part = _gates_partial_h(inp_h, w_h)
    inp_x = _seg_x(t2, el_t, fg, fs, ts)
    h_new, c_new = _gates_final_x(inp_x, w_x, part, b_all, c_node)
    return h_new, c_new


# speedup vs baseline: 1.1670x; 1.0127x over previous
import jax
import jax.numpy as jnp
from jax import lax
from jax.experimental import pallas as pl
from jax.experimental.pallas import tpu as pltpu
from jax.experimental.pallas import tpu_sc as plsc

N = 10000
E = 160000
NT = 16
NC = 2
EPT = E // NT
B = 128
EH = 64
NB = 80
G = 16
EPT_PAD = NB * B
E_PAD = NT * EPT_PAD
TRASH = N
ACC_STRIPE = 632
ACC_ROWS = NT * ACC_STRIPE
CW = 128
LPC = CW // 16
HCn = 1024 // CW
XCn = 256 // CW
K_TOT = 2 * XCn + 2 * HCn
D_GATE = 4096
NBLK = 1000


def _mm_t_body(a_ref, w_ref, b_ref, o_ref):
    o_ref[0] = (
        jnp.dot(a_ref[...], w_ref[0], preferred_element_type=jnp.float32)
        + b_ref[0, 0]
    )


def _mm_t(e_token, w_tok, b_link):
    return pl.pallas_call(
        _mm_t_body,
        grid=(N // NBLK, XCn),
        in_specs=[
            pl.BlockSpec((NBLK, 256), lambda i, c: (i, 0)),
            pl.BlockSpec((1, 256, CW), lambda i, c: (c, 0, 0)),
            pl.BlockSpec((1, 1, CW), lambda i, c: (c, 0, 0)),
        ],
        out_specs=pl.BlockSpec((1, NBLK, CW), lambda i, c: (c, i, 0)),
        out_shape=jax.ShapeDtypeStruct((XCn, N, CW), jnp.float32),
    )(e_token, w_tok.reshape(256, XCn, CW).transpose(1, 0, 2),
      b_link.reshape(XCn, 1, CW))


def _mm_el_body(a_ref, w_ref, o_ref):
    o_ref[0] = jnp.dot(a_ref[...], w_ref[0], preferred_element_type=jnp.float32)


def _mm_el(e_link_pad, w_el):
    eblk = 2048
    return pl.pallas_call(
        _mm_el_body,
        grid=(E_PAD // eblk, XCn),
        in_specs=[
            pl.BlockSpec((eblk, 128), lambda i, c: (i, 0)),
            pl.BlockSpec((1, 128, CW), lambda i, c: (c, 0, 0)),
        ],
        out_specs=pl.BlockSpec((1, eblk, CW), lambda i, c: (c, i, 0)),
        out_shape=jax.ShapeDtypeStruct((XCn, E_PAD, CW), jnp.float32),
    )(e_link_pad, w_el.reshape(128, XCn, CW).transpose(1, 0, 2))


def _sc_core(mode, h2, t2, el_t, fg, fs, tg, ts, out,
             igblk, iscblk, bufA0, bufA1, bufEl, acc,
             semA, semB, semS0, semS1):
    cid = lax.axis_index("c")
    sid = lax.axis_index("s")
    bufAs = (bufA0, bufA1)
    sems = (semA, semB)
    ssems = (semS0, semS1)

    def zero_bufEl():
        def _zrow(r, carry):
            for cc in range(LPC):
                bufEl[r, pl.ds(cc * 16, 16)] = jnp.zeros((16,), jnp.float32)
            return carry
        lax.fori_loop(0, EH, _zrow, 0)

    def zero_acc():
        base = sid * ACC_STRIPE
        nfull = ACC_STRIPE // EH
        for q in range(nfull):
            pltpu.sync_copy(bufEl, acc.at[pl.ds(base + q * EH, EH)])
        rem = ACC_STRIPE % EH
        if rem:
            pltpu.sync_copy(bufEl.at[pl.ds(0, rem)],
                            acc.at[pl.ds(base + nfull * EH, rem)])

    def flush(kk):
        lo = sid * ACC_STRIPE

        @pl.when(sid != NT - 1)
        def _():
            pltpu.sync_copy(acc.at[pl.ds(lo, ACC_STRIPE)],
                            out.at[kk, pl.ds(lo, ACC_STRIPE)])

        @pl.when(sid == NT - 1)
        def _():
            last = N - (NT - 1) * ACC_STRIPE
            pltpu.sync_copy(acc.at[pl.ds((NT - 1) * ACC_STRIPE, last)],
                            out.at[kk, pl.ds((NT - 1) * ACC_STRIPE, last)])

    def fetch_and_start(tab, r, q):
        pltpu.make_async_copy(tab.at[igblk.at[r]], bufAs[q], sems[q]).start()

    def gather_wait(tab, r, q):
        pltpu.make_async_copy(tab.at[igblk.at[r]], bufAs[q], sems[q]).wait()

    def scatter_start(r, q):
        pltpu.make_async_copy(
            bufAs[q], acc.at[iscblk.at[r]], ssems[q]).start(add=True)

    def scatter_wait(r, p):
        pltpu.make_async_copy(
            bufAs[p], acc.at[iscblk.at[r]], ssems[p]).wait()

    def do_pass(kk, tab, ig, isc, mult, off, xc):
        zero_acc()
        plsc.subcore_barrier()

        def blkbody(blk, carry):
            pltpu.sync_copy(ig.at[sid, pl.ds(blk * G, G)], igblk)
            pltpu.sync_copy(isc.at[sid, pl.ds(blk * G, G)], iscblk)

            def _scale(r, c):
                for cc in range(B // 16):
                    sl = pl.ds(cc * 16, 16)
                    igblk[r, sl] = igblk[r, sl] * mult + off
                return c
            lax.fori_loop(0, G, _scale, 0)
            fetch_and_start(tab, 0, 0)

            def step(i, carry2):
                for q in range(2):
                    r = 2 * i + q
                    gather_wait(tab, r, q)
                    if xc is None:
                        if q == 1:
                            scatter_wait(2 * i, 0)
                        else:
                            @pl.when(i > 0)
                            def _():
                                scatter_wait(2 * i - 1, 1)

                    @pl.when(r + 1 < G)
                    def _():
                        fetch_and_start(tab, r + 1, 1 - q)

                    if xc is None:
                        scatter_start(r, q)
                    else:
                        bb = blk * G + r
                        for hh in range(B // EH):
                            pltpu.sync_copy(
                                el_t.at[xc, pl.ds(
                                    sid * EPT_PAD + bb * B + hh * EH, EH)],
                                bufEl)

                            def trow(rr, c2):
                                for cc in range(LPC):
                                    sl = pl.ds(cc * 16, 16)
                                    v = bufEl[rr, sl] + bufAs[q][hh * EH + rr, sl]
                                    ex = jnp.exp(v * 2.0)
                                    bufEl[rr, sl] = 1.0 - 2.0 / (ex + 1.0)
                                return c2
                            lax.fori_loop(0, EH, trow, 0)
                            pltpu.sync_copy(
                                bufEl,
                                acc.at[iscblk.at[r, pl.ds(hh * EH, EH)]],
                                add=True)
                return carry2
            lax.fori_loop(0, G // 2, step, 0)
            if xc is None:
                scatter_wait(G - 1, 1)
            return carry
        lax.fori_loop(0, NB // G, blkbody, 0)
        plsc.subcore_barrier()
        flush(kk)
        return 0

    zero_bufEl()
    if mode == "h":
        def h_in(p, c):
            chunk = cid * (HCn // NC) + p
            return do_pass(chunk, h2, fg, ts, HCn, chunk, None)

        def h_out(p, c):
            chunk = cid * (HCn // NC) + p
            return do_pass(HCn + chunk, h2, tg, fs, HCn, chunk, None)

        lax.fori_loop(0, HCn // NC, h_in, 0)
        lax.fori_loop(0, HCn // NC, h_out, 0)
    else:
        for p in range(XCn // NC):
            xc = cid * (XCn // NC) + p
            do_pass(xc, t2, fg, ts, 1, xc * N, xc)
            zero_bufEl()
            do_pass(XCn + xc, t2, fg, fs, 1, xc * N, xc)
            zero_bufEl()


def _sc_body_h(h2, fg, fs, tg, ts, out, *rest):
    _sc_core("h", h2, None, None, fg, fs, tg, ts, out, *rest)


def _sc_body_x(t2, el_t, fg, fs, ts, out, *rest):
    _sc_core("x", None, t2, el_t, fg, fs, None, ts, out, *rest)


def _sc_scratch():
    return [
        pltpu.VMEM((G, B), jnp.int32),
        pltpu.VMEM((G, B), jnp.int32),
        pltpu.VMEM((B, CW), jnp.float32),
        pltpu.VMEM((B, CW), jnp.float32),
        pltpu.VMEM((EH, CW), jnp.float32),
        pltpu.VMEM_SHARED((ACC_ROWS, CW), jnp.float32),
        pltpu.SemaphoreType.DMA,
        pltpu.SemaphoreType.DMA,
        pltpu.SemaphoreType.DMA,
        pltpu.SemaphoreType.DMA,
    ]


def _seg_h(h2, fg, fs, tg, ts):
    mesh = plsc.VectorSubcoreMesh(core_axis_name="c", subcore_axis_name="s")
    return pl.kernel(
        _sc_body_h,
        out_type=jax.ShapeDtypeStruct((2 * HCn, N, CW), jnp.float32),
        mesh=mesh,
        scratch_types=_sc_scratch(),
    )(h2, fg, fs, tg, ts)


def _seg_x(t2, el_t, fg, fs, ts):
    mesh = plsc.VectorSubcoreMesh(core_axis_name="c", subcore_axis_name="s")
    return pl.kernel(
        _sc_body_x,
        out_type=jax.ShapeDtypeStruct((2 * XCn, N, CW), jnp.float32),
        mesh=mesh,
        scratch_types=_sc_scratch(),
    )(t2, el_t, fg, fs, ts)


def _gate_h_body(inp_ref, w_ref, part_out, acc):
    k = pl.program_id(1)

    @pl.when(k == 0)
    def _():
        acc[...] = jnp.zeros_like(acc)

    acc[...] += jnp.dot(inp_ref[0], w_ref[0], preferred_element_type=jnp.float32)

    @pl.when(k == 2 * HCn - 1)
    def _():
        part_out[...] = acc[...]


def _gates_partial_h(inp_h, w_h):
    return pl.pallas_call(
        _gate_h_body,
        grid=(N // NBLK, 2 * HCn),
        in_specs=[
            pl.BlockSpec((1, NBLK, CW), lambda i, k: (k, i, 0)),
            pl.BlockSpec((1, CW, D_GATE), lambda i, k: (k, 0, 0)),
        ],
        out_specs=pl.BlockSpec((NBLK, D_GATE), lambda i, k: (i, 0)),
        out_shape=jax.ShapeDtypeStruct((N, D_GATE), jnp.float32),
        scratch_shapes=[pltpu.VMEM((NBLK, D_GATE), jnp.float32)],
    )(inp_h, w_h)


def _gate_x_body(inp_ref, w_ref, part_ref, b_ref, c_ref, h_out, c_out, acc):
    k = pl.program_id(1)

    @pl.when(k == 0)
    def _():
        acc[...] = part_ref[...]

    acc[...] += jnp.dot(inp_ref[0], w_ref[0], preferred_element_type=jnp.float32)

    @pl.when(k == 2 * XCn - 1)
    def _():
        g = acc[...] + b_ref[...]
        gi = g[:, 0:1024]
        go = g[:, 1024:2048]
        gf = g[:, 2048:3072]
        gu = g[:, 3072:4096]
        si = 1.0 / (1.0 + jnp.exp(-gi))
        so = 1.0 / (1.0 + jnp.exp(-go))
        sf = 1.0 / (1.0 + jnp.exp(-gf))
        u = jnp.tanh(gu)
        c2 = sf * c_ref[...] + si * u
        c_out[...] = c2
        h_out[...] = so * jnp.tanh(c2)


def _gates_final_x(inp_x, w_x, part, b_all, c_node):
    nb = 400
    return pl.pallas_call(
        _gate_x_body,
        grid=(N // nb, 2 * XCn),
        in_specs=[
            pl.BlockSpec((1, nb, CW), lambda i, k: (k, i, 0)),
            pl.BlockSpec((1, CW, D_GATE), lambda i, k: (k, 0, 0)),
            pl.BlockSpec((nb, D_GATE), lambda i, k: (i, 0)),
            pl.BlockSpec((1, D_GATE), lambda i, k: (0, 0)),
            pl.BlockSpec((nb, 1024), lambda i, k: (i, 0)),
        ],
        out_specs=[
            pl.BlockSpec((nb, 1024), lambda i, k: (i, 0)),
            pl.BlockSpec((nb, 1024), lambda i, k: (i, 0)),
        ],
        out_shape=[
            jax.ShapeDtypeStruct((N, 1024), jnp.float32),
            jax.ShapeDtypeStruct((N, 1024), jnp.float32),
        ],
        scratch_shapes=[pltpu.VMEM((nb, D_GATE), jnp.float32)],
    )(inp_x, w_x, part, b_all, c_node)


def kernel(h_node, c_node, e_link, e_token, i_from, i_to,
           W_link, b_link, W_i, b_i, W_o, b_o, W_f, b_f, W_u, b_u):
    w_el = W_link[:128]
    w_tok = W_link[128:]
    w_all = jnp.concatenate([W_i, W_o, W_f, W_u], axis=1).reshape(K_TOT, CW, D_GATE)
    w_x = w_all[:2 * XCn]
    w_h = w_all[2 * XCn:]
    b_all = jnp.concatenate([b_i, b_o, b_f, b_u]).reshape(1, D_GATE)

    def pad_to(ix, dummy):
        a2 = ix.reshape(NT, EPT)
        pad = jnp.full((NT, EPT_PAD - EPT), dummy, jnp.int32)
        return jnp.concatenate([a2, pad], axis=1).reshape(NT, NB, B)

    fg = pad_to(i_from, 0)
    tg = pad_to(i_to, 0)
    fs = pad_to(i_from, TRASH)
    ts = pad_to(i_to, TRASH)

    e_link_pad = jnp.pad(
        e_link.reshape(NT, EPT, 128), ((0, 0), (0, EPT_PAD - EPT), (0, 0))
    ).reshape(E_PAD, 128)

    h2 = h_node.reshape(N * HCn, CW)
    inp_h = _seg_h(h2, fg, fs, tg, ts)
    t2 = _mm_t(e_token, w_tok, b_link).reshape(N * XCn, CW)
    el_t = _mm_el(e_link_pad, w_el)
    part = _gates_partial_h(inp_h, w_h)
    inp_x = _seg_x(t2, el_t, fg, fs, ts)
    h_new, c_new = _gates_final_x(inp_x, w_x, part, b_all, c_node)
    return h_new, c_new
